# baseline (speedup 1.0000x reference)
import functools

import jax
import jax.numpy as jnp
from jax import lax
from jax.experimental import pallas as pl
from jax.experimental.pallas import tpu as pltpu
from jax.experimental.pallas import tpu_sc as plsc

F = 64
B = 16384
NC = 2
NS = 16
L = 16
NW = NC * NS
BPW = B // NW
CHUNK = 128
NCH = BPW // CHUNK
UNROLL = 4

_mesh = plsc.VectorSubcoreMesh(core_axis_name="c", subcore_axis_name="s")


@functools.partial(
    pl.kernel,
    out_type=jax.ShapeDtypeStruct((B,), jnp.float32),
    mesh=_mesh,
    compiler_params=pltpu.CompilerParams(
        use_tc_tiling_on_sc=False, needs_layout_passes=False),
    scratch_types=[
        pltpu.VMEM((NCH, CHUNK), jnp.int32),
        pltpu.VMEM((NCH, CHUNK), jnp.int32),
        pltpu.VMEM((BPW, F), jnp.float32),
        pltpu.VMEM((BPW, F), jnp.float32),
        pltpu.VMEM((BPW * L,), jnp.float32),
        pltpu.VMEM((BPW,), jnp.float32),
        pltpu.SemaphoreType.DMA,
    ],
)
def _sc_dot(users_hbm, items_hbm, ut_hbm, it_hbm, out_hbm,
            uidx, iidx, urows, irows, psum, outv, sem):
    wid = lax.axis_index("s") * NC + lax.axis_index("c")
    base = wid * BPW

    for j in range(NCH):
        pltpu.sync_copy(users_hbm.at[pl.ds(base + j * CHUNK, CHUNK)], uidx.at[j])
        pltpu.sync_copy(items_hbm.at[pl.ds(base + j * CHUNK, CHUNK)], iidx.at[j])

    copies = []
    for j in range(NCH):
        copies.append(pltpu.async_copy(
            ut_hbm.at[uidx.at[j]], urows.at[pl.ds(j * CHUNK, CHUNK)], sem))
        copies.append(pltpu.async_copy(
            it_hbm.at[iidx.at[j]], irows.at[pl.ds(j * CHUNK, CHUNK)], sem))
    for c in copies:
        c.wait()

    def rows_body(i, carry):
        for k in range(UNROLL):
            r = i * UNROLL + k
            acc = urows[r, pl.ds(0, L)] * irows[r, pl.ds(0, L)]
            for c0 in range(1, F // L):
                acc = acc + urows[r, pl.ds(c0 * L, L)] * irows[r, pl.ds(c0 * L, L)]
            psum[pl.ds(r * L, L)] = acc
        return carry

    lax.fori_loop(0, BPW // UNROLL, rows_body, 0)

    lanes = lax.iota(jnp.int32, L) * L

    def red_body(g, carry):
        bi = lanes + g * (L * L)
        acc = plsc.load_gather(psum, [bi])
        for p in range(1, L):
            acc = acc + plsc.load_gather(psum, [bi + p])
        outv[pl.ds(g * L, L)] = acc
        return carry

    lax.fori_loop(0, BPW // L, red_body, 0)

    pltpu.sync_copy(outv, out_hbm.at[pl.ds(base, BPW)])


def kernel(users, items, user_table, item_table):
    return _sc_dot(users.astype(jnp.int32), items.astype(jnp.int32),
                   user_table, item_table)
